# baseline (speedup 1.0000x reference)
import functools

import jax
import jax.numpy as jnp
from jax import lax
from jax.experimental import pallas as pl
from jax.experimental.pallas import tpu as pltpu
from jax.experimental.pallas import tpu_sc as plsc

N_U = 5000
N_I = 5000
NN = N_U + N_I
DD = 128
EE = 320000

NC = 2
NS = 16
NW = NC * NS
BK = 128
NBUF = 2
B0 = 112
B1 = 48
BLK_TOT = NS * (B0 + B1)
EP = BLK_TOT * BK

NP = 10240
ROWS_PER_TILE = NP // NS
ZR = 32

@functools.cache
def _get_sc_scatter():
    mesh = plsc.VectorSubcoreMesh(
        core_axis_name="c", subcore_axis_name="s",
        num_cores=NC, num_subcores=NS)

    @functools.partial(
        pl.kernel,
        out_type=jax.ShapeDtypeStruct((NC, NP, DD), jnp.float32),
        mesh=mesh,
        scratch_types=[
            pltpu.VMEM((NBUF, BK), jnp.int32),
            pltpu.VMEM((NBUF, BK), jnp.int32),
            pltpu.VMEM((NBUF, BK * 16), jnp.float32),
            pltpu.VMEM((NBUF, BK, DD), jnp.float32),
            pltpu.VMEM((ZR, DD), jnp.float32),
            pltpu.VMEM_SHARED((NP, DD), jnp.float32),
            pltpu.SemaphoreType.DMA((NBUF,)),
            pltpu.SemaphoreType.DMA((NBUF,)),
            pltpu.SemaphoreType.DMA((NBUF,)),
            pltpu.SemaphoreType.DMA((NBUF,)),
        ],
    )
    def _sc_scatter(ego_hbm, col_hbm, row_hbm, w_hbm, out_hbm,
                    colr, rowr, wbuf, gbuf, zbuf, acc,
                    gsem, rsem, wsem, ssem):
        c = lax.axis_index("c")
        s = lax.axis_index("s")
        wid = c * NS + s
        nblocks = jnp.where(c == 0, B0, B1)

        zero16 = jnp.zeros((16,), jnp.float32)
        for r in range(ZR):
            for g in range(DD // 16):
                zbuf[r, pl.ds(g * 16, 16)] = zero16
        row0 = s * ROWS_PER_TILE
        def _zcopy(i, _):
            pltpu.sync_copy(zbuf, acc.at[pl.ds(row0 + i * ZR, ZR)])
            return ()
        lax.fori_loop(0, ROWS_PER_TILE // ZR, _zcopy, (), unroll=False)
        plsc.subcore_barrier()

        def _start(j, p):
            pltpu.sync_copy(col_hbm.at[wid, j], colr.at[p])
            pltpu.async_copy(ego_hbm.at[colr.at[p]], gbuf.at[p], gsem.at[p])
            pltpu.async_copy(row_hbm.at[wid, j], rowr.at[p], rsem.at[p])
            pltpu.async_copy(w_hbm.at[wid, j], wbuf.at[p], wsem.at[p])

        def _consume(j, p):
            pltpu.make_async_copy(
                ego_hbm.at[colr.at[p]], gbuf.at[p], gsem.at[p]).wait()
            pltpu.make_async_copy(
                w_hbm.at[0, 0], wbuf.at[p], wsem.at[p]).wait()
            def _scale(e, _):
                we = wbuf[p, pl.ds(e * 16, 16)]
                for g in range(DD // 16):
                    gbuf[p, e, pl.ds(g * 16, 16)] = (
                        gbuf[p, e, pl.ds(g * 16, 16)] * we)
                return ()
            lax.fori_loop(0, BK, _scale, (), unroll=4)
            pltpu.make_async_copy(
                row_hbm.at[0, 0], rowr.at[p], rsem.at[p]).wait()
            pltpu.async_copy(gbuf.at[p], acc.at[rowr.at[p]], ssem.at[p],
                             add=True)

        def _wait_scatter(p):
            pltpu.make_async_copy(
                gbuf.at[p], acc.at[rowr.at[p]], ssem.at[p]).wait()

        _start(0, 0)

        def _pair(h, _):
            for p in range(2):
                j = 2 * h + p
                q = 1 - p
                @pl.when(2 * h + p + 1 < nblocks)
                def _():
                    @pl.when(2 * h + p >= 1)
                    def _():
                        _wait_scatter(q)
                    _start(j + 1, q)
                _consume(j, p)
            return ()
        lax.fori_loop(0, nblocks // 2, _pair, (), unroll=False)

        for p in range(2):
            _wait_scatter(p)
        plsc.subcore_barrier()

        pltpu.sync_copy(acc.at[pl.ds(row0, ROWS_PER_TILE)],
                        out_hbm.at[c, pl.ds(row0, ROWS_PER_TILE)])

    return _sc_scatter


_RB = 1000


def _pre_body(u_ref, mwt_ref, mb_ref, w1t_ref, w2t_ref, rb_ref, out_ref):
    u = u_ref[...]
    sh = jnp.maximum(
        jnp.dot(u, mwt_ref[...], preferred_element_type=jnp.float32)
        + mb_ref[...], 0.0)
    o = (jnp.dot(u, w1t_ref[...], preferred_element_type=jnp.float32)
         + jnp.dot(sh, w2t_ref[...], preferred_element_type=jnp.float32)
         + rb_ref[...])
    out_ref[...] = jnp.maximum(o, 0.0)


def _leaky(x):
    return jnp.where(x >= 0, x, 0.01 * x)


def _layer_body(p0_ref, p1_ref, ego_ref, gwt_ref, gb_ref, bwt_ref, bb_ref,
                ego_out_ref, norm_out_ref):
    side = p0_ref[...] + p1_ref[...]
    ego = ego_ref[...]
    sum_emb = _leaky(
        jnp.dot(side, gwt_ref[...], preferred_element_type=jnp.float32)
        + gb_ref[...])
    bi_emb = _leaky(
        jnp.dot(ego * side, bwt_ref[...], preferred_element_type=jnp.float32)
        + bb_ref[...])
    e = sum_emb + bi_emb
    nrm = jnp.sqrt(jnp.sum(e * e, axis=1, keepdims=True))
    ego_out_ref[...] = e
    norm_out_ref[...] = e / jnp.maximum(nrm, 1e-12)


def _pre_call(u, mwt, mb, w1t, w2t, rb):
    grid = N_U // _RB
    full = pl.BlockSpec((DD, DD), lambda i: (0, 0))
    bias = pl.BlockSpec((1, DD), lambda i: (0, 0))
    return pl.pallas_call(
        _pre_body,
        grid=(grid,),
        in_specs=[pl.BlockSpec((_RB, DD), lambda i: (i, 0)),
                  full, bias, full, full, bias],
        out_specs=pl.BlockSpec((_RB, DD), lambda i: (i, 0)),
        out_shape=jax.ShapeDtypeStruct((N_U, DD), jnp.float32),
    )(u, mwt, mb, w1t, w2t, rb)


def _layer_call(p0, p1, ego, gwt, gb, bwt, bb):
    grid = NN // _RB
    full = pl.BlockSpec((DD, DD), lambda i: (0, 0))
    bias = pl.BlockSpec((1, DD), lambda i: (0, 0))
    rows = pl.BlockSpec((_RB, DD), lambda i: (i, 0))
    return pl.pallas_call(
        _layer_body,
        grid=(grid,),
        in_specs=[rows, rows, rows, full, bias, full, bias],
        out_specs=[rows, rows],
        out_shape=[jax.ShapeDtypeStruct((NN, DD), jnp.float32),
                   jax.ShapeDtypeStruct((NN, DD), jnp.float32)],
    )(p0, p1, ego, gwt, gb, bwt, bb)


def kernel(edge_index, edge_weight, u2e_r, u2e_t, item_emb, mlp_W, mlp_b,
           mlp_r_W, mlp_r_b, mlp_t_W, mlp_t_b,
           GC_W0, GC_b0, GC_W1, GC_b1, Bi_W0, Bi_b0, Bi_W1, Bi_b1):
    row = edge_index[0].astype(jnp.int32)
    col = edge_index[1].astype(jnp.int32)
    pad = EP - EE

    def _worker_layout(xp, lanes):
        cut = NS * B0 * BK * lanes
        a = xp[:cut].reshape(NS, B0, BK * lanes)
        b = xp[cut:].reshape(NS, B1, BK * lanes)
        b = jnp.pad(b, ((0, 0), (0, B0 - B1), (0, 0)))
        return jnp.concatenate([a, b], axis=0)

    col_p = _worker_layout(jnp.pad(col, (0, pad)), 1)
    row_p = _worker_layout(jnp.pad(row, (0, pad)), 1)
    w_b = _worker_layout(
        jnp.broadcast_to(jnp.pad(edge_weight, (0, pad))[:, None],
                         (EP, 16)).reshape(EP * 16), 16)

    u_new = _pre_call(u2e_r[:N_U], mlp_W.T, mlp_b[None],
                      mlp_r_W[:, :DD].T, mlp_r_W[:, DD:].T, mlp_r_b[None])
    ego = jnp.concatenate([u_new, item_emb], axis=0)

    outs = [ego]
    for (GW, Gb, BW, Bb) in ((GC_W0, GC_b0, Bi_W0, Bi_b0),
                             (GC_W1, GC_b1, Bi_W1, Bi_b1)):
        partials = _get_sc_scatter()(ego, col_p, row_p, w_b)
        ego, nrm = _layer_call(partials[0, :NN], partials[1, :NN], ego,
                               GW.T, Gb[None], BW.T, Bb[None])
        outs.append(nrm)

    all_emb = jnp.concatenate(outs, axis=1)
    return all_emb[:N_U], all_emb[N_U:]


# speedup vs baseline: 1.1931x; 1.1931x over previous
import functools

import jax
import jax.numpy as jnp
from jax import lax
from jax.experimental import pallas as pl
from jax.experimental.pallas import tpu as pltpu
from jax.experimental.pallas import tpu_sc as plsc

N_U = 5000
N_I = 5000
NN = N_U + N_I
DD = 128
EE = 320000

NC = 2
NS = 16
NW = NC * NS
BK = 128
NBUF = 2
BLOCKS = 80
EP = NW * BLOCKS * BK

NP = 10240
ROWS_PER_TILE = NP // NS
ZR = 32

@functools.cache
def _get_sc_scatter():
    mesh = plsc.VectorSubcoreMesh(
        core_axis_name="c", subcore_axis_name="s",
        num_cores=NC, num_subcores=NS)

    @functools.partial(
        pl.kernel,
        out_type=jax.ShapeDtypeStruct((NC, NP, DD), jnp.float32),
        mesh=mesh,
        scratch_types=[
            pltpu.VMEM((NBUF, BK), jnp.int32),
            pltpu.VMEM((NBUF, BK), jnp.int32),
            pltpu.VMEM((NBUF, BK * 16), jnp.float32),
            pltpu.VMEM((NBUF, BK, DD), jnp.float32),
            pltpu.VMEM((ZR, DD), jnp.float32),
            pltpu.VMEM_SHARED((NP, DD), jnp.float32),
            pltpu.SemaphoreType.DMA((NBUF,)),
            pltpu.SemaphoreType.DMA((NBUF,)),
            pltpu.SemaphoreType.DMA((NBUF,)),
            pltpu.SemaphoreType.DMA((NBUF,)),
        ],
    )
    def _sc_scatter(ego_hbm, col_hbm, row_hbm, w_hbm, out_hbm,
                    colr, rowr, wbuf, gbuf, zbuf, acc,
                    gsem, rsem, wsem, ssem):
        c = lax.axis_index("c")
        s = lax.axis_index("s")
        wid = c * NS + s
        nblocks = BLOCKS

        zero16 = jnp.zeros((16,), jnp.float32)
        for r in range(ZR):
            for g in range(DD // 16):
                zbuf[r, pl.ds(g * 16, 16)] = zero16
        row0 = s * ROWS_PER_TILE
        def _zcopy(i, _):
            pltpu.sync_copy(zbuf, acc.at[pl.ds(row0 + i * ZR, ZR)])
            return ()
        lax.fori_loop(0, ROWS_PER_TILE // ZR, _zcopy, (), unroll=False)
        plsc.subcore_barrier()

        def _start(j, p):
            pltpu.sync_copy(col_hbm.at[wid, j], colr.at[p])
            pltpu.async_copy(ego_hbm.at[colr.at[p]], gbuf.at[p], gsem.at[p])
            pltpu.async_copy(row_hbm.at[wid, j], rowr.at[p], rsem.at[p])
            pltpu.async_copy(w_hbm.at[wid, j], wbuf.at[p], wsem.at[p])

        def _consume(j, p):
            pltpu.make_async_copy(
                ego_hbm.at[colr.at[p]], gbuf.at[p], gsem.at[p]).wait()
            pltpu.make_async_copy(
                w_hbm.at[0, 0], wbuf.at[p], wsem.at[p]).wait()
            def _scale(e, _):
                we = wbuf[p, pl.ds(e * 16, 16)]
                for g in range(DD // 16):
                    gbuf[p, e, pl.ds(g * 16, 16)] = (
                        gbuf[p, e, pl.ds(g * 16, 16)] * we)
                return ()
            lax.fori_loop(0, BK, _scale, (), unroll=4)
            pltpu.make_async_copy(
                row_hbm.at[0, 0], rowr.at[p], rsem.at[p]).wait()
            pltpu.async_copy(gbuf.at[p], acc.at[rowr.at[p]], ssem.at[p],
                             add=True)

        def _wait_scatter(p):
            pltpu.make_async_copy(
                gbuf.at[p], acc.at[rowr.at[p]], ssem.at[p]).wait()

        _start(0, 0)

        def _pair(h, _):
            for p in range(2):
                j = 2 * h + p
                q = 1 - p
                @pl.when(2 * h + p + 1 < nblocks)
                def _():
                    @pl.when(2 * h + p >= 1)
                    def _():
                        _wait_scatter(q)
                    _start(j + 1, q)
                _consume(j, p)
            return ()
        lax.fori_loop(0, nblocks // 2, _pair, (), unroll=False)

        for p in range(2):
            _wait_scatter(p)
        plsc.subcore_barrier()

        pltpu.sync_copy(acc.at[pl.ds(row0, ROWS_PER_TILE)],
                        out_hbm.at[c, pl.ds(row0, ROWS_PER_TILE)])

    return _sc_scatter


_RB = 1000


def _pre_body(u_ref, mwt_ref, mb_ref, w1t_ref, w2t_ref, rb_ref, out_ref):
    u = u_ref[...]
    sh = jnp.maximum(
        jnp.dot(u, mwt_ref[...], preferred_element_type=jnp.float32)
        + mb_ref[...], 0.0)
    o = (jnp.dot(u, w1t_ref[...], preferred_element_type=jnp.float32)
         + jnp.dot(sh, w2t_ref[...], preferred_element_type=jnp.float32)
         + rb_ref[...])
    out_ref[...] = jnp.maximum(o, 0.0)


def _leaky(x):
    return jnp.where(x >= 0, x, 0.01 * x)


def _layer_body(p0_ref, p1_ref, ego_ref, gwt_ref, gb_ref, bwt_ref, bb_ref,
                ego_out_ref, norm_out_ref):
    side = p0_ref[0] + p1_ref[0]
    ego = ego_ref[...]
    sum_emb = _leaky(
        jnp.dot(side, gwt_ref[...], preferred_element_type=jnp.float32)
        + gb_ref[...])
    bi_emb = _leaky(
        jnp.dot(ego * side, bwt_ref[...], preferred_element_type=jnp.float32)
        + bb_ref[...])
    e = sum_emb + bi_emb
    nrm = jnp.sqrt(jnp.sum(e * e, axis=1, keepdims=True))
    ego_out_ref[...] = e
    norm_out_ref[...] = e / jnp.maximum(nrm, 1e-12)


_WB = 4096


def _wbcast_body(w_ref, out_ref):
    lane = jax.lax.broadcasted_iota(jnp.int32, (8, 128), 1)
    sub = jax.lax.broadcasted_iota(jnp.int32, (8, 128), 0)
    sel = (lane // 16 == sub).astype(jnp.float32)
    out_ref[...] = jnp.dot(w_ref[...], sel,
                           preferred_element_type=jnp.float32,
                           precision=jax.lax.Precision.HIGHEST)


def _wbcast_call(w_pad):
    rows = EP // 8
    return pl.pallas_call(
        _wbcast_body,
        grid=(rows // _WB,),
        in_specs=[pl.BlockSpec((_WB, 8), lambda i: (i, 0))],
        out_specs=pl.BlockSpec((_WB, 128), lambda i: (i, 0)),
        out_shape=jax.ShapeDtypeStruct((rows, 128), jnp.float32),
    )(w_pad.reshape(rows, 8))


def _pre_call(u, mwt, mb, w1t, w2t, rb):
    grid = N_U // _RB
    full = pl.BlockSpec((DD, DD), lambda i: (0, 0))
    bias = pl.BlockSpec((1, DD), lambda i: (0, 0))
    return pl.pallas_call(
        _pre_body,
        grid=(grid,),
        in_specs=[pl.BlockSpec((_RB, DD), lambda i: (i, 0)),
                  full, bias, full, full, bias],
        out_specs=pl.BlockSpec((_RB, DD), lambda i: (i, 0)),
        out_shape=jax.ShapeDtypeStruct((N_U, DD), jnp.float32),
    )(u, mwt, mb, w1t, w2t, rb)


def _layer_call(partials, ego, gwt, gb, bwt, bb):
    grid = NN // _RB
    full = pl.BlockSpec((DD, DD), lambda i: (0, 0))
    bias = pl.BlockSpec((1, DD), lambda i: (0, 0))
    rows = pl.BlockSpec((_RB, DD), lambda i: (i, 0))
    p0 = pl.BlockSpec((1, _RB, DD), lambda i: (0, i, 0))
    p1 = pl.BlockSpec((1, _RB, DD), lambda i: (1, i, 0))
    return pl.pallas_call(
        _layer_body,
        grid=(grid,),
        in_specs=[p0, p1, rows, full, bias, full, bias],
        out_specs=[rows, rows],
        out_shape=[jax.ShapeDtypeStruct((NN, DD), jnp.float32),
                   jax.ShapeDtypeStruct((NN, DD), jnp.float32)],
    )(partials, partials, ego, gwt, gb, bwt, bb)


def kernel(edge_index, edge_weight, u2e_r, u2e_t, item_emb, mlp_W, mlp_b,
           mlp_r_W, mlp_r_b, mlp_t_W, mlp_t_b,
           GC_W0, GC_b0, GC_W1, GC_b1, Bi_W0, Bi_b0, Bi_W1, Bi_b1):
    row = edge_index[0].astype(jnp.int32)
    col = edge_index[1].astype(jnp.int32)
    pad = EP - EE
    col_p = jnp.pad(col, (0, pad)).reshape(NW, BLOCKS, BK)
    row_p = jnp.pad(row, (0, pad)).reshape(NW, BLOCKS, BK)
    w_b = _wbcast_call(jnp.pad(edge_weight, (0, pad))).reshape(
        NW, BLOCKS, BK * 16)

    u_new = _pre_call(u2e_r[:N_U], mlp_W.T, mlp_b[None],
                      mlp_r_W[:, :DD].T, mlp_r_W[:, DD:].T, mlp_r_b[None])
    ego = jnp.concatenate([u_new, item_emb], axis=0)

    outs = [ego]
    for (GW, Gb, BW, Bb) in ((GC_W0, GC_b0, Bi_W0, Bi_b0),
                             (GC_W1, GC_b1, Bi_W1, Bi_b1)):
        partials = _get_sc_scatter()(ego, col_p, row_p, w_b)
        ego, nrm = _layer_call(partials, ego, GW.T, Gb[None], BW.T, Bb[None])
        outs.append(nrm)

    all_emb = jnp.concatenate(outs, axis=1)
    return all_emb[:N_U], all_emb[N_U:]


# speedup vs baseline: 1.2366x; 1.0364x over previous
import functools

import jax
import jax.numpy as jnp
from jax import lax
from jax.experimental import pallas as pl
from jax.experimental.pallas import tpu as pltpu
from jax.experimental.pallas import tpu_sc as plsc

N_U = 5000
N_I = 5000
NN = N_U + N_I
DD = 128
EE = 320000

NC = 2
NS = 16
NW = NC * NS
BK = 128
NBUF = 2
BLOCKS = 80
EP = NW * BLOCKS * BK

NP = 10240
ROWS_PER_TILE = NP // NS
ZR = 32

@functools.cache
def _get_sc_scatter():
    mesh = plsc.VectorSubcoreMesh(
        core_axis_name="c", subcore_axis_name="s",
        num_cores=NC, num_subcores=NS)

    @functools.partial(
        pl.kernel,
        out_type=jax.ShapeDtypeStruct((NC, NP, DD), jnp.float32),
        mesh=mesh,
        scratch_types=[
            pltpu.VMEM((NBUF, BK), jnp.int32),
            pltpu.VMEM((NBUF, BK), jnp.int32),
            pltpu.VMEM((NBUF, BK // 8, 128), jnp.float32),
            pltpu.VMEM((NBUF, BK, DD), jnp.float32),
            pltpu.VMEM((ZR, DD), jnp.float32),
            pltpu.VMEM_SHARED((NP, DD), jnp.float32),
            pltpu.SemaphoreType.DMA((NBUF,)),
            pltpu.SemaphoreType.DMA((NBUF,)),
            pltpu.SemaphoreType.DMA((NBUF,)),
            pltpu.SemaphoreType.DMA((NBUF,)),
        ],
    )
    def _sc_scatter(ego_hbm, ei_hbm, w_hbm, out_hbm,
                    colr, rowr, wbuf, gbuf, zbuf, acc,
                    gsem, rsem, wsem, ssem):
        c = lax.axis_index("c")
        s = lax.axis_index("s")
        wid = c * NS + s
        nblocks = BLOCKS

        with jax.named_scope("acc_zero"):
            zero16 = jnp.zeros((16,), jnp.float32)
            for r in range(ZR):
                for g in range(DD // 16):
                    zbuf[r, pl.ds(g * 16, 16)] = zero16
            row0 = s * ROWS_PER_TILE
            def _zcopy(i, _):
                pltpu.sync_copy(zbuf, acc.at[pl.ds(row0 + i * ZR, ZR)])
                return ()
            lax.fori_loop(0, ROWS_PER_TILE // ZR, _zcopy, (), unroll=False)
            plsc.subcore_barrier()

        def _start(j, p):
            pltpu.sync_copy(ei_hbm.at[1, wid, j], colr.at[p])
            pltpu.async_copy(ego_hbm.at[colr.at[p]], gbuf.at[p], gsem.at[p])
            pltpu.async_copy(ei_hbm.at[0, wid, j], rowr.at[p], rsem.at[p])
            pltpu.async_copy(w_hbm.at[wid, pl.ds(j * (BK // 8), BK // 8)],
                             wbuf.at[p], wsem.at[p])

        def _consume(j, p):
            pltpu.make_async_copy(
                ego_hbm.at[colr.at[p]], gbuf.at[p], gsem.at[p]).wait()
            pltpu.make_async_copy(
                w_hbm.at[0, pl.ds(0, BK // 8)], wbuf.at[p],
                wsem.at[p]).wait()
            def _scale(e, _):
                we = wbuf[p, e // 8, pl.ds((e % 8) * 16, 16)]
                for g in range(DD // 16):
                    gbuf[p, e, pl.ds(g * 16, 16)] = (
                        gbuf[p, e, pl.ds(g * 16, 16)] * we)
                return ()
            lax.fori_loop(0, BK, _scale, (), unroll=4)
            pltpu.make_async_copy(
                ei_hbm.at[0, 0, 0], rowr.at[p], rsem.at[p]).wait()
            pltpu.async_copy(gbuf.at[p], acc.at[rowr.at[p]], ssem.at[p],
                             add=True)

        def _wait_scatter(p):
            pltpu.make_async_copy(
                gbuf.at[p], acc.at[rowr.at[p]], ssem.at[p]).wait()

        with jax.named_scope("prime"):
            _start(0, 0)

        def _pair(h, _):
            for p in range(2):
                j = 2 * h + p
                q = 1 - p
                @pl.when(2 * h + p + 1 < nblocks)
                def _():
                    @pl.when(2 * h + p >= 1)
                    def _():
                        _wait_scatter(q)
                    _start(j + 1, q)
                _consume(j, p)
            return ()
        with jax.named_scope("edge_loop"):
            lax.fori_loop(0, nblocks // 2, _pair, (), unroll=False)

        with jax.named_scope("drain_writeback"):
            for p in range(2):
                _wait_scatter(p)
            plsc.subcore_barrier()

            pltpu.sync_copy(acc.at[pl.ds(row0, ROWS_PER_TILE)],
                            out_hbm.at[c, pl.ds(row0, ROWS_PER_TILE)])

    return _sc_scatter


_RB = 1000


def _pre_body(u_ref, mwt_ref, mb_ref, w1t_ref, w2t_ref, rb_ref, out_ref):
    u = u_ref[...]
    sh = jnp.maximum(
        jnp.dot(u, mwt_ref[...], preferred_element_type=jnp.float32)
        + mb_ref[...], 0.0)
    o = (jnp.dot(u, w1t_ref[...], preferred_element_type=jnp.float32)
         + jnp.dot(sh, w2t_ref[...], preferred_element_type=jnp.float32)
         + rb_ref[...])
    out_ref[...] = jnp.maximum(o, 0.0)


def _leaky(x):
    return jnp.where(x >= 0, x, 0.01 * x)


def _layer_body(p0_ref, p1_ref, ego_ref, gwt_ref, gb_ref, bwt_ref, bb_ref,
                ego_out_ref, norm_out_ref):
    side = p0_ref[0] + p1_ref[0]
    ego = ego_ref[...]
    sum_emb = _leaky(
        jnp.dot(side, gwt_ref[...], preferred_element_type=jnp.float32)
        + gb_ref[...])
    bi_emb = _leaky(
        jnp.dot(ego * side, bwt_ref[...], preferred_element_type=jnp.float32)
        + bb_ref[...])
    e = sum_emb + bi_emb
    nrm = jnp.sqrt(jnp.sum(e * e, axis=1, keepdims=True))
    ego_out_ref[...] = e
    norm_out_ref[...] = e / jnp.maximum(nrm, 1e-12)


def _wbcast_body(w_ref, out_ref):
    lane = jax.lax.broadcasted_iota(jnp.int32, (8, 128), 1)
    sub = jax.lax.broadcasted_iota(jnp.int32, (8, 128), 0)
    sel = (lane // 16 == sub).astype(jnp.float32)
    out_ref[0] = jnp.dot(w_ref[...], sel,
                         preferred_element_type=jnp.float32,
                         precision=jax.lax.Precision.HIGHEST)


def _wbcast_call(w_pad):
    rows = EP // 8
    rpw = rows // NW
    return pl.pallas_call(
        _wbcast_body,
        grid=(NW,),
        in_specs=[pl.BlockSpec((rpw, 8), lambda i: (i, 0))],
        out_specs=pl.BlockSpec((1, rpw, 128), lambda i: (i, 0, 0)),
        out_shape=jax.ShapeDtypeStruct((NW, rpw, 128), jnp.float32),
    )(w_pad.reshape(rows, 8))


def _pre_call(u, mwt, mb, w1t, w2t, rb):
    grid = N_U // _RB
    full = pl.BlockSpec((DD, DD), lambda i: (0, 0))
    bias = pl.BlockSpec((1, DD), lambda i: (0, 0))
    return pl.pallas_call(
        _pre_body,
        grid=(grid,),
        in_specs=[pl.BlockSpec((_RB, DD), lambda i: (i, 0)),
                  full, bias, full, full, bias],
        out_specs=pl.BlockSpec((_RB, DD), lambda i: (i, 0)),
        out_shape=jax.ShapeDtypeStruct((N_U, DD), jnp.float32),
    )(u, mwt, mb, w1t, w2t, rb)


def _layer_call(partials, ego, gwt, gb, bwt, bb):
    grid = NN // _RB
    full = pl.BlockSpec((DD, DD), lambda i: (0, 0))
    bias = pl.BlockSpec((1, DD), lambda i: (0, 0))
    rows = pl.BlockSpec((_RB, DD), lambda i: (i, 0))
    p0 = pl.BlockSpec((1, _RB, DD), lambda i: (0, i, 0))
    p1 = pl.BlockSpec((1, _RB, DD), lambda i: (1, i, 0))
    return pl.pallas_call(
        _layer_body,
        grid=(grid,),
        in_specs=[p0, p1, rows, full, bias, full, bias],
        out_specs=[rows, rows],
        out_shape=[jax.ShapeDtypeStruct((NN, DD), jnp.float32),
                   jax.ShapeDtypeStruct((NN, DD), jnp.float32)],
    )(partials, partials, ego, gwt, gb, bwt, bb)


def kernel(edge_index, edge_weight, u2e_r, u2e_t, item_emb, mlp_W, mlp_b,
           mlp_r_W, mlp_r_b, mlp_t_W, mlp_t_b,
           GC_W0, GC_b0, GC_W1, GC_b1, Bi_W0, Bi_b0, Bi_W1, Bi_b1):
    row = edge_index[0].astype(jnp.int32)
    col = edge_index[1].astype(jnp.int32)
    pad = EP - EE
    ei_p = jnp.pad(jnp.stack([row, col]), ((0, 0), (0, pad))).reshape(
        2, NW, BLOCKS, BK)
    w_b = _wbcast_call(jnp.pad(edge_weight, (0, pad)))

    u_new = _pre_call(u2e_r[:N_U], mlp_W.T, mlp_b[None],
                      mlp_r_W[:, :DD].T, mlp_r_W[:, DD:].T, mlp_r_b[None])
    ego = jnp.concatenate([u_new, item_emb], axis=0)

    outs = [ego]
    for (GW, Gb, BW, Bb) in ((GC_W0, GC_b0, Bi_W0, Bi_b0),
                             (GC_W1, GC_b1, Bi_W1, Bi_b1)):
        partials = _get_sc_scatter()(ego, ei_p, w_b)
        ego, nrm = _layer_call(partials, ego, GW.T, Gb[None], BW.T, Bb[None])
        outs.append(nrm)

    all_emb = jnp.concatenate(outs, axis=1)
    return all_emb[:N_U], all_emb[N_U:]


# speedup vs baseline: 1.4888x; 1.2039x over previous
import functools

import jax
import jax.numpy as jnp
from jax import lax
from jax.experimental import pallas as pl
from jax.experimental.pallas import tpu as pltpu
from jax.experimental.pallas import tpu_sc as plsc

N_U = 5000
N_I = 5000
NN = N_U + N_I
DD = 128
EE = 320000

NC = 2
NS = 16
NW = NC * NS
BK = 128
NBUF = 2
B0 = 144
B1 = 16
TOTB = NS * (B0 + B1)
EP = TOTB * BK

NP = 10240
ROWS_PER_TILE = NP // NS

@functools.cache
def _get_sc_scatter():
    mesh = plsc.VectorSubcoreMesh(
        core_axis_name="c", subcore_axis_name="s",
        num_cores=NC, num_subcores=NS)

    @functools.partial(
        pl.kernel,
        out_type=jax.ShapeDtypeStruct((NC, NP, DD), jnp.float32),
        mesh=mesh,
        scratch_types=[
            pltpu.VMEM((NBUF, BK), jnp.int32),
            pltpu.VMEM((NBUF, BK), jnp.int32),
            pltpu.VMEM((NBUF, BK // 8, 128), jnp.float32),
            pltpu.VMEM((NBUF, BK, DD), jnp.float32),
            pltpu.VMEM_SHARED((NP, DD), jnp.float32),
            pltpu.SemaphoreType.DMA((NBUF,)),
            pltpu.SemaphoreType.DMA((NBUF,)),
            pltpu.SemaphoreType.DMA((NBUF,)),
            pltpu.SemaphoreType.DMA((NBUF,)),
        ],
    )
    def _sc_scatter(ego_hbm, ei_hbm, w_hbm, out_hbm,
                    colr, rowr, wbuf, gbuf, acc,
                    gsem, rsem, wsem, ssem):
        c = lax.axis_index("c")
        s = lax.axis_index("s")
        wid = c * NS + s
        nblocks = jnp.where(c == 0, B0, B1)
        bstart = jnp.where(c == 0, s * B0, NS * B0 + s * B1)

        with jax.named_scope("acc_zero"):
            zero16 = jnp.zeros((16,), jnp.float32)
            def _zfill(r, _):
                for g in range(DD // 16):
                    gbuf[0, r, pl.ds(g * 16, 16)] = zero16
                return ()
            lax.fori_loop(0, BK, _zfill, (), unroll=4)
            row0 = s * ROWS_PER_TILE
            def _zcopy(i, _):
                pltpu.sync_copy(gbuf.at[0],
                                acc.at[pl.ds(row0 + i * BK, BK)])
                return ()
            lax.fori_loop(0, ROWS_PER_TILE // BK, _zcopy, (), unroll=False)
            plsc.subcore_barrier()

        def _start(j, p):
            pltpu.sync_copy(ei_hbm.at[1, bstart + j], colr.at[p])
            pltpu.async_copy(ego_hbm.at[colr.at[p]], gbuf.at[p], gsem.at[p])
            pltpu.async_copy(ei_hbm.at[0, bstart + j], rowr.at[p],
                             rsem.at[p])
            pltpu.async_copy(w_hbm.at[bstart + j], wbuf.at[p], wsem.at[p])

        def _consume(j, p):
            pltpu.make_async_copy(
                ego_hbm.at[colr.at[p]], gbuf.at[p], gsem.at[p]).wait()
            pltpu.make_async_copy(
                w_hbm.at[0], wbuf.at[p], wsem.at[p]).wait()
            def _scale(e, _):
                we = wbuf[p, e // 8, pl.ds((e % 8) * 16, 16)]
                for g in range(DD // 16):
                    gbuf[p, e, pl.ds(g * 16, 16)] = (
                        gbuf[p, e, pl.ds(g * 16, 16)] * we)
                return ()
            lax.fori_loop(0, BK, _scale, (), unroll=4)
            pltpu.make_async_copy(
                ei_hbm.at[0, 0], rowr.at[p], rsem.at[p]).wait()
            pltpu.async_copy(gbuf.at[p], acc.at[rowr.at[p]], ssem.at[p],
                             add=True)

        def _wait_scatter(p):
            pltpu.make_async_copy(
                gbuf.at[p], acc.at[rowr.at[p]], ssem.at[p]).wait()

        with jax.named_scope("prime"):
            _start(0, 0)

        def _pair(h, _):
            for p in range(2):
                j = 2 * h + p
                q = 1 - p
                @pl.when(2 * h + p + 1 < nblocks)
                def _():
                    @pl.when(2 * h + p >= 1)
                    def _():
                        _wait_scatter(q)
                    _start(j + 1, q)
                _consume(j, p)
            return ()
        with jax.named_scope("edge_loop"):
            lax.fori_loop(0, nblocks // 2, _pair, (), unroll=False)

        with jax.named_scope("drain_writeback"):
            for p in range(2):
                _wait_scatter(p)
            plsc.subcore_barrier()

            pltpu.sync_copy(acc.at[pl.ds(row0, ROWS_PER_TILE)],
                            out_hbm.at[c, pl.ds(row0, ROWS_PER_TILE)])

    return _sc_scatter


_RB = 1000


def _pre_body(u_ref, mwt_ref, mb_ref, w1t_ref, w2t_ref, rb_ref, out_ref):
    u = u_ref[...]
    sh = jnp.maximum(
        jnp.dot(u, mwt_ref[...], preferred_element_type=jnp.float32)
        + mb_ref[...], 0.0)
    o = (jnp.dot(u, w1t_ref[...], preferred_element_type=jnp.float32)
         + jnp.dot(sh, w2t_ref[...], preferred_element_type=jnp.float32)
         + rb_ref[...])
    out_ref[...] = jnp.maximum(o, 0.0)


def _leaky(x):
    return jnp.where(x >= 0, x, 0.01 * x)


def _layer_body(p0_ref, p1_ref, ego_ref, gwt_ref, gb_ref, bwt_ref, bb_ref,
                ego_out_ref, norm_out_ref):
    side = p0_ref[0] + p1_ref[0]
    ego = ego_ref[...]
    sum_emb = _leaky(
        jnp.dot(side, gwt_ref[...], preferred_element_type=jnp.float32)
        + gb_ref[...])
    bi_emb = _leaky(
        jnp.dot(ego * side, bwt_ref[...], preferred_element_type=jnp.float32)
        + bb_ref[...])
    e = sum_emb + bi_emb
    nrm = jnp.sqrt(jnp.sum(e * e, axis=1, keepdims=True))
    ego_out_ref[...] = e
    norm_out_ref[...] = e / jnp.maximum(nrm, 1e-12)


_WGB = 128


def _wbcast_call(w_pad):
    rows = EP // 8
    rpg = _WGB * BK // 8

    def body(w_ref, out_ref):
        lane = jax.lax.broadcasted_iota(jnp.int32, (8, 128), 1)
        sub = jax.lax.broadcasted_iota(jnp.int32, (8, 128), 0)
        sel = (lane // 16 == sub).astype(jnp.float32)
        r = jnp.dot(w_ref[...], sel, preferred_element_type=jnp.float32,
                    precision=jax.lax.Precision.HIGHEST)
        out_ref[...] = r.reshape(_WGB, BK // 8, 128)

    return pl.pallas_call(
        body,
        grid=(TOTB // _WGB,),
        in_specs=[pl.BlockSpec((rpg, 8), lambda i: (i, 0))],
        out_specs=pl.BlockSpec((_WGB, BK // 8, 128), lambda i: (i, 0, 0)),
        out_shape=jax.ShapeDtypeStruct((TOTB, BK // 8, 128), jnp.float32),
    )(w_pad.reshape(rows, 8))


def _pre_call(u, mwt, mb, w1t, w2t, rb):
    grid = N_U // _RB
    full = pl.BlockSpec((DD, DD), lambda i: (0, 0))
    bias = pl.BlockSpec((1, DD), lambda i: (0, 0))
    return pl.pallas_call(
        _pre_body,
        grid=(grid,),
        in_specs=[pl.BlockSpec((_RB, DD), lambda i: (i, 0)),
                  full, bias, full, full, bias],
        out_specs=pl.BlockSpec((_RB, DD), lambda i: (i, 0)),
        out_shape=jax.ShapeDtypeStruct((N_U, DD), jnp.float32),
    )(u, mwt, mb, w1t, w2t, rb)


def _layer_call(partials, ego, gwt, gb, bwt, bb):
    grid = NN // _RB
    full = pl.BlockSpec((DD, DD), lambda i: (0, 0))
    bias = pl.BlockSpec((1, DD), lambda i: (0, 0))
    rows = pl.BlockSpec((_RB, DD), lambda i: (i, 0))
    p0 = pl.BlockSpec((1, _RB, DD), lambda i: (0, i, 0))
    p1 = pl.BlockSpec((1, _RB, DD), lambda i: (1, i, 0))
    return pl.pallas_call(
        _layer_body,
        grid=(grid,),
        in_specs=[p0, p1, rows, full, bias, full, bias],
        out_specs=[rows, rows],
        out_shape=[jax.ShapeDtypeStruct((NN, DD), jnp.float32),
                   jax.ShapeDtypeStruct((NN, DD), jnp.float32)],
    )(partials, partials, ego, gwt, gb, bwt, bb)


def kernel(edge_index, edge_weight, u2e_r, u2e_t, item_emb, mlp_W, mlp_b,
           mlp_r_W, mlp_r_b, mlp_t_W, mlp_t_b,
           GC_W0, GC_b0, GC_W1, GC_b1, Bi_W0, Bi_b0, Bi_W1, Bi_b1):
    row = edge_index[0].astype(jnp.int32)
    col = edge_index[1].astype(jnp.int32)
    pad = EP - EE
    ei_p = jnp.pad(jnp.stack([row, col]), ((0, 0), (0, pad))).reshape(
        2, TOTB, BK)
    w_b = _wbcast_call(jnp.pad(edge_weight, (0, pad)))

    u_new = _pre_call(u2e_r[:N_U], mlp_W.T, mlp_b[None],
                      mlp_r_W[:, :DD].T, mlp_r_W[:, DD:].T, mlp_r_b[None])
    ego = jnp.concatenate([u_new, item_emb], axis=0)

    outs = [ego]
    for (GW, Gb, BW, Bb) in ((GC_W0, GC_b0, Bi_W0, Bi_b0),
                             (GC_W1, GC_b1, Bi_W1, Bi_b1)):
        partials = _get_sc_scatter()(ego, ei_p, w_b)
        ego, nrm = _layer_call(partials, ego, GW.T, Gb[None], BW.T, Bb[None])
        outs.append(nrm)

    all_emb = jnp.concatenate(outs, axis=1)
    return all_emb[:N_U], all_emb[N_U:]
